# baseline (speedup 1.0000x reference)
import jax
import jax.numpy as jnp
from jax import lax
from jax.experimental import pallas as pl
from jax.experimental.pallas import tpu as pltpu

B = 1024
N_CTX = 4
D = 512
ROWS = 77
P_PRE, P_MID, P_SUF = 5, 2, 62
OFF_CLS = P_PRE
OFF_MID = OFF_CLS + N_CTX
OFF_CLO = OFF_MID + P_MID
OFF_SUF = OFF_CLO + N_CTX

EPB = 16
STEPS = B // EPB
RING = 6


def _asm_body(lbl_s, clo_s, cls_hbm, clo_hbm, tmpl_ref, out_hbm,
              stage, cls_v, clo_v, out_sem, g_sem):
    i = pl.program_id(0)
    b0 = i * EPB
    cur = lax.rem(i, RING)
    base = cur * EPB

    copies = []
    for e in range(EPB):
        c1 = pltpu.make_async_copy(cls_hbm.at[lbl_s[b0 + e]], cls_v.at[e],
                                   g_sem.at[0, e])
        c2 = pltpu.make_async_copy(clo_hbm.at[clo_s[b0 + e]], clo_v.at[e],
                                   g_sem.at[1, e])
        c1.start()
        c2.start()
        copies.append((c1, c2))

    @pl.when(i < RING)
    def _prefill():
        for e in range(EPB):
            stage[pl.ds(base + e, 1)] = tmpl_ref[...].reshape(1, ROWS, D)

    for r in range(RING):
        @pl.when(jnp.logical_and(i >= RING, cur == r))
        def _recycle(r=r):
            pltpu.make_async_copy(stage.at[pl.ds(r * EPB, EPB)],
                                  out_hbm.at[pl.ds(0, EPB)],
                                  out_sem.at[r]).wait()

    for e in range(EPB):
        c1, c2 = copies[e]
        c1.wait()
        c2.wait()
        stage[pl.ds(base + e, 1), OFF_CLS:OFF_CLS + N_CTX] = (
            cls_v[e].reshape(1, N_CTX, D))
        stage[pl.ds(base + e, 1), OFF_CLO:OFF_CLO + N_CTX] = (
            clo_v[e].reshape(1, N_CTX, D))

    for r in range(RING):
        @pl.when(cur == r)
        def _fire(r=r):
            pltpu.make_async_copy(stage.at[pl.ds(r * EPB, EPB)],
                                  out_hbm.at[pl.ds(b0, EPB)],
                                  out_sem.at[r]).start()

    @pl.when(i == STEPS - 1)
    def _drain():
        for r in range(RING):
            pltpu.make_async_copy(stage.at[pl.ds(r * EPB, EPB)],
                                  out_hbm.at[pl.ds(0, EPB)],
                                  out_sem.at[r]).wait()


@jax.jit
def _prompt_assemble(label, cloth_label, cls_ctx, clo_ctx, tmpl_full):
    grid_spec = pltpu.PrefetchScalarGridSpec(
        num_scalar_prefetch=2,
        grid=(STEPS,),
        in_specs=[
            pl.BlockSpec(memory_space=pltpu.MemorySpace.HBM),
            pl.BlockSpec(memory_space=pltpu.MemorySpace.HBM),
            pl.BlockSpec((ROWS, D), lambda i, lbl, clo: (0, 0)),
        ],
        out_specs=pl.BlockSpec(memory_space=pltpu.MemorySpace.HBM),
        scratch_shapes=[
            pltpu.VMEM((RING * EPB, ROWS, D), jnp.float32),
            pltpu.VMEM((EPB, N_CTX, D), jnp.float32),
            pltpu.VMEM((EPB, N_CTX, D), jnp.float32),
            pltpu.SemaphoreType.DMA((RING,)),
            pltpu.SemaphoreType.DMA((2, EPB)),
        ],
    )
    return pl.pallas_call(
        _asm_body,
        grid_spec=grid_spec,
        out_shape=jax.ShapeDtypeStruct((B, ROWS, D), jnp.float32),
        compiler_params=pltpu.CompilerParams(
            dimension_semantics=("arbitrary",)),
    )(label, cloth_label, cls_ctx, clo_ctx, tmpl_full)


def kernel(label, cloth_label, cls_ctx, cls_cloth_ctx,
           token_prefix, token_middle, token_suffix):
    zeros4 = jnp.zeros((N_CTX, D), jnp.float32)
    tmpl_full = jnp.concatenate(
        [token_prefix.reshape(P_PRE, D), zeros4,
         token_middle.reshape(P_MID, D), zeros4,
         token_suffix.reshape(P_SUF, D)], axis=0)
    out = _prompt_assemble(label.astype(jnp.int32),
                           cloth_label.astype(jnp.int32),
                           cls_ctx, cls_cloth_ctx, tmpl_full)
    return (out, 17)


# speedup vs baseline: 2.7155x; 2.7155x over previous
import jax
import jax.numpy as jnp
from jax.experimental import pallas as pl
from jax.experimental.pallas import tpu as pltpu

B = 1024
N_CTX = 4
D = 512
ROWS = 77
P_PRE, P_MID, P_SUF = 5, 2, 62
OFF_CLS = P_PRE
OFF_MID = OFF_CLS + N_CTX
OFF_CLO = OFF_MID + P_MID
OFF_SUF = OFF_CLO + N_CTX

_GATHER_POS = list(range(OFF_CLS, OFF_CLS + N_CTX)) + \
              list(range(OFF_CLO, OFF_CLO + N_CTX))
_BCAST_POS = [p for p in range(ROWS) if p not in _GATHER_POS]
_ORDER = _BCAST_POS + _GATHER_POS
N_BCAST = len(_BCAST_POS)

GI_STEPS = 64
EPG = B // GI_STEPS
N_CLOTH = 1000


def _asm_body(lbl_s, clo_s, ord_s, cls_hbm, clo_hbm, tmpl_blk, out_ref,
              cls_st, clo_st, g_sem):
    i = pl.program_id(0)

    @pl.when(i < GI_STEPS)
    def _issue():
        for e in range(EPG):
            b = i * EPG + e
            pltpu.make_async_copy(cls_hbm.at[lbl_s[b]], cls_st.at[b],
                                  g_sem.at[0]).start()
            pltpu.make_async_copy(clo_hbm.at[clo_s[b]], clo_st.at[b],
                                  g_sem.at[1]).start()

    @pl.when(i < N_BCAST)
    def _bcast():
        out_ref[0] = jnp.broadcast_to(tmpl_blk[0], (B, D))

    @pl.when(i == N_BCAST)
    def _drain():
        pltpu.make_async_copy(cls_hbm.at[pl.ds(0, B)], cls_st,
                              g_sem.at[0]).wait()
        pltpu.make_async_copy(clo_hbm.at[pl.ds(0, N_CLOTH)],
                              clo_st.at[pl.ds(0, N_CLOTH)],
                              g_sem.at[1]).wait()
        pltpu.make_async_copy(clo_hbm.at[pl.ds(0, B - N_CLOTH)],
                              clo_st.at[pl.ds(N_CLOTH, B - N_CLOTH)],
                              g_sem.at[1]).wait()

    for k in range(N_CTX):
        @pl.when(i == N_BCAST + k)
        def _gcls(k=k):
            out_ref[0] = cls_st[:, k, :]

        @pl.when(i == N_BCAST + N_CTX + k)
        def _gclo(k=k):
            out_ref[0] = clo_st[:, k, :]


@jax.jit
def _prompt_assemble(label, cloth_label, order, cls_ctx, clo_ctx, tmpl3):
    grid_spec = pltpu.PrefetchScalarGridSpec(
        num_scalar_prefetch=3,
        grid=(ROWS,),
        in_specs=[
            pl.BlockSpec(memory_space=pltpu.MemorySpace.HBM),
            pl.BlockSpec(memory_space=pltpu.MemorySpace.HBM),
            pl.BlockSpec((1, 1, D), lambda i, lbl, clo, o: (o[i], 0, 0)),
        ],
        out_specs=pl.BlockSpec((1, B, D), lambda i, lbl, clo, o: (o[i], 0, 0)),
        scratch_shapes=[
            pltpu.VMEM((B, N_CTX, D), jnp.float32),
            pltpu.VMEM((B, N_CTX, D), jnp.float32),
            pltpu.SemaphoreType.DMA((2,)),
        ],
    )
    return pl.pallas_call(
        _asm_body,
        grid_spec=grid_spec,
        out_shape=jax.ShapeDtypeStruct((ROWS, B, D), jnp.float32),
        compiler_params=pltpu.CompilerParams(
            dimension_semantics=("arbitrary",)),
    )(label, cloth_label, order, cls_ctx, clo_ctx, tmpl3)


def kernel(label, cloth_label, cls_ctx, cls_cloth_ctx,
           token_prefix, token_middle, token_suffix):
    zeros4 = jnp.zeros((N_CTX, D), jnp.float32)
    tmpl3 = jnp.concatenate(
        [token_prefix.reshape(P_PRE, D), zeros4,
         token_middle.reshape(P_MID, D), zeros4,
         token_suffix.reshape(P_SUF, D)], axis=0).reshape(ROWS, 1, D)
    order = jnp.asarray(_ORDER, dtype=jnp.int32)
    out77 = _prompt_assemble(label.astype(jnp.int32),
                             cloth_label.astype(jnp.int32),
                             order, cls_ctx, cls_cloth_ctx, tmpl3)
    return (jnp.transpose(out77, (1, 0, 2)), 17)


# speedup vs baseline: 3.2855x; 1.2099x over previous
import jax
import jax.numpy as jnp
from jax.experimental import pallas as pl
from jax.experimental.pallas import tpu as pltpu

B = 1024
N_CTX = 4
D = 512
ROWS = 77
P_PRE, P_MID, P_SUF = 5, 2, 62
OFF_CLS = P_PRE
OFF_MID = OFF_CLS + N_CTX
OFF_CLO = OFF_MID + P_MID
OFF_SUF = OFF_CLO + N_CTX

PPB = 7
NBLK = ROWS // PPB
_ORDER = [3, 4, 5, 6, 7, 8, 9, 10, 0, 1, 2]
_FIRST_GATHER_STEP = _ORDER.index(0)

GI_STEPS = _FIRST_GATHER_STEP
EPG = B // GI_STEPS
N_CLOTH = 1000


def _asm_body(lbl_s, clo_s, ord_s, cls_hbm, clo_hbm, tmpl_ref, out_ref,
              cls_st, clo_st, g_sem):
    i = pl.program_id(0)

    @pl.when(i < GI_STEPS)
    def _issue():
        for e in range(EPG):
            b = i * EPG + e
            pltpu.make_async_copy(cls_hbm.at[lbl_s[b]], cls_st.at[b],
                                  g_sem.at[0]).start()
            pltpu.make_async_copy(clo_hbm.at[clo_s[b]], clo_st.at[b],
                                  g_sem.at[1]).start()

    @pl.when(i == _FIRST_GATHER_STEP)
    def _drain():
        pltpu.make_async_copy(cls_hbm.at[pl.ds(0, B)], cls_st,
                              g_sem.at[0]).wait()
        pltpu.make_async_copy(clo_hbm.at[pl.ds(0, N_CLOTH)],
                              clo_st.at[pl.ds(0, N_CLOTH)],
                              g_sem.at[1]).wait()
        pltpu.make_async_copy(clo_hbm.at[pl.ds(0, B - N_CLOTH)],
                              clo_st.at[pl.ds(N_CLOTH, B - N_CLOTH)],
                              g_sem.at[1]).wait()

    for j, blk in enumerate(_ORDER):
        @pl.when(i == j)
        def _emit(blk=blk):
            for q in range(PPB):
                p = PPB * blk + q
                if OFF_CLS <= p < OFF_MID:
                    out_ref[q] = cls_st[:, p - OFF_CLS, :]
                elif OFF_CLO <= p < OFF_SUF:
                    out_ref[q] = clo_st[:, p - OFF_CLO, :]
                else:
                    out_ref[q] = jnp.broadcast_to(tmpl_ref[p], (B, D))


@jax.jit
def _prompt_assemble(label, cloth_label, order, cls_ctx, clo_ctx, tmpl_full):
    grid_spec = pltpu.PrefetchScalarGridSpec(
        num_scalar_prefetch=3,
        grid=(NBLK,),
        in_specs=[
            pl.BlockSpec(memory_space=pltpu.MemorySpace.HBM),
            pl.BlockSpec(memory_space=pltpu.MemorySpace.HBM),
            pl.BlockSpec((ROWS, D), lambda i, lbl, clo, o: (0, 0)),
        ],
        out_specs=pl.BlockSpec((PPB, B, D),
                               lambda i, lbl, clo, o: (o[i], 0, 0)),
        scratch_shapes=[
            pltpu.VMEM((B, N_CTX, D), jnp.float32),
            pltpu.VMEM((B, N_CTX, D), jnp.float32),
            pltpu.SemaphoreType.DMA((2,)),
        ],
    )
    return pl.pallas_call(
        _asm_body,
        grid_spec=grid_spec,
        out_shape=jax.ShapeDtypeStruct((ROWS, B, D), jnp.float32),
        compiler_params=pltpu.CompilerParams(
            dimension_semantics=("arbitrary",),
            vmem_limit_bytes=50 * 1024 * 1024),
    )(label, cloth_label, order, cls_ctx, clo_ctx, tmpl_full)


def kernel(label, cloth_label, cls_ctx, cls_cloth_ctx,
           token_prefix, token_middle, token_suffix):
    zeros4 = jnp.zeros((N_CTX, D), jnp.float32)
    tmpl_full = jnp.concatenate(
        [token_prefix.reshape(P_PRE, D), zeros4,
         token_middle.reshape(P_MID, D), zeros4,
         token_suffix.reshape(P_SUF, D)], axis=0)
    order = jnp.asarray(_ORDER, dtype=jnp.int32)
    out77 = _prompt_assemble(label.astype(jnp.int32),
                             cloth_label.astype(jnp.int32),
                             order, cls_ctx, cls_cloth_ctx, tmpl_full)
    return (jnp.transpose(out77, (1, 0, 2)), 17)
